# baseline (speedup 1.0000x reference)
import functools

import jax
import jax.numpy as jnp
from jax import lax
from jax.experimental import pallas as pl
from jax.experimental.pallas import tpu as pltpu
from jax.experimental.pallas import tpu_sc as plsc

NUM_WORKERS = 32
CHUNK = 128


def _num_token_tc(X_num, Wt, b2):
    B, K = X_num.shape
    D = Wt.shape[1]
    Bb = 2048

    def body(x_ref, w_ref, b_ref, o_ref):
        o_ref[...] = (
            jnp.dot(x_ref[...], w_ref[...], preferred_element_type=jnp.float32)
            + b_ref[...]
        )

    return pl.pallas_call(
        body,
        grid=(B // Bb,),
        in_specs=[
            pl.BlockSpec((Bb, K), lambda i: (i, 0)),
            pl.BlockSpec((K, D), lambda i: (0, 0)),
            pl.BlockSpec((1, D), lambda i: (0, 0)),
        ],
        out_specs=pl.BlockSpec((Bb, D), lambda i: (i, 0)),
        out_shape=jax.ShapeDtypeStruct((B, D), jnp.float32),
    )(X_num, Wt, b2)


def _sc_tokens(tbl_flat, gidx, oidx, num_token, noidx, out_rows):
    D = tbl_flat.shape[1]
    n_cat = gidx.shape[0]
    n_num = noidx.shape[0]
    assert n_cat % (NUM_WORKERS * CHUNK) == 0
    assert n_num % (NUM_WORKERS * CHUNK) == 0
    cat_per_w = n_cat // NUM_WORKERS
    num_per_w = n_num // NUM_WORKERS
    cat_chunks = cat_per_w // CHUNK
    num_chunks = num_per_w // CHUNK

    mesh = plsc.VectorSubcoreMesh(core_axis_name="c", subcore_axis_name="s")

    @functools.partial(
        pl.kernel,
        mesh=mesh,
        compiler_params=pltpu.CompilerParams(use_tc_tiling_on_sc=False),
        out_type=jax.ShapeDtypeStruct((out_rows, D), jnp.float32),
        scratch_types=[
            pltpu.VMEM((CHUNK,), jnp.int32),
            pltpu.VMEM((CHUNK,), jnp.int32),
            pltpu.VMEM((CHUNK, D), jnp.float32),
            pltpu.SemaphoreType.DMA,
        ],
    )
    def k(tbl_hbm, gidx_hbm, oidx_hbm, num_hbm, noidx_hbm, out_hbm,
          gidx_v, oidx_v, rows_v, sem):
        wid = lax.axis_index("s") * 2 + lax.axis_index("c")
        cat_base = wid * cat_per_w
        num_base = wid * num_per_w

        @pl.loop(0, cat_chunks)
        def _(j):
            start = cat_base + j * CHUNK
            pltpu.sync_copy(gidx_hbm.at[pl.ds(start, CHUNK)], gidx_v)
            pltpu.sync_copy(oidx_hbm.at[pl.ds(start, CHUNK)], oidx_v)
            pltpu.async_copy(tbl_hbm.at[gidx_v], rows_v, sem).wait()
            pltpu.sync_copy(rows_v, out_hbm.at[oidx_v])

        @pl.loop(0, num_chunks)
        def _(j):
            start = num_base + j * CHUNK
            pltpu.sync_copy(noidx_hbm.at[pl.ds(start, CHUNK)], oidx_v)
            pltpu.sync_copy(num_hbm.at[pl.ds(start, CHUNK)], rows_v)
            pltpu.sync_copy(rows_v, out_hbm.at[oidx_v])

    return k(tbl_flat, gidx, oidx, num_token, noidx)


def kernel(X_num, X_cat, W_num, b_num, tables):
    B, _ = X_num.shape
    NC = X_cat.shape[1]
    V = tables.shape[1]
    D = tables.shape[2]
    T = NC + 1

    num_token = _num_token_tc(X_num, W_num.T, b_num[None, :])

    x_cat = X_cat.astype(jnp.int32)
    field_off = (jnp.arange(NC, dtype=jnp.int32) * V)[None, :]
    gidx = (x_cat + field_off).reshape(B * NC)
    row_base = (jnp.arange(B, dtype=jnp.int32) * T)[:, None]
    oidx = (row_base + 1 + jnp.arange(NC, dtype=jnp.int32)[None, :]).reshape(B * NC)
    noidx = jnp.arange(B, dtype=jnp.int32) * T

    tbl_flat = tables.reshape(NC * V, D)
    out_flat = _sc_tokens(tbl_flat, gidx, oidx, num_token, noidx, B * T)
    return out_flat.reshape(B, T, D)


# speedup vs baseline: 1.1159x; 1.1159x over previous
import functools

import jax
import jax.numpy as jnp
from jax import lax
from jax.experimental import pallas as pl
from jax.experimental.pallas import tpu as pltpu
from jax.experimental.pallas import tpu_sc as plsc

NUM_WORKERS = 32
CHUNK = 128


def _num_token_tc(X_num, Wt, b2):
    B, K = X_num.shape
    D = Wt.shape[1]
    Bb = 2048

    def body(x_ref, w_ref, b_ref, o_ref):
        o_ref[...] = (
            jnp.dot(x_ref[...], w_ref[...], preferred_element_type=jnp.float32)
            + b_ref[...]
        )

    return pl.pallas_call(
        body,
        grid=(B // Bb,),
        in_specs=[
            pl.BlockSpec((Bb, K), lambda i: (i, 0)),
            pl.BlockSpec((K, D), lambda i: (0, 0)),
            pl.BlockSpec((1, D), lambda i: (0, 0)),
        ],
        out_specs=pl.BlockSpec((Bb, D), lambda i: (i, 0)),
        out_shape=jax.ShapeDtypeStruct((B, D), jnp.float32),
    )(X_num, Wt, b2)


GROUP = 8


def _sc_tokens(tbl_flat, gidx, oidx, num_token, noidx, out_rows):
    D = tbl_flat.shape[1]
    n_cat = gidx.shape[0]
    n_num = noidx.shape[0]
    assert n_cat % (NUM_WORKERS * CHUNK) == 0
    assert n_num % (NUM_WORKERS * CHUNK) == 0
    cat_per_w = n_cat // NUM_WORKERS
    num_per_w = n_num // NUM_WORKERS
    cat_chunks = cat_per_w // CHUNK
    num_chunks = num_per_w // CHUNK
    assert cat_chunks % GROUP == 0

    gidx3 = gidx.reshape(NUM_WORKERS, cat_chunks, CHUNK)
    oidx3 = oidx.reshape(NUM_WORKERS, cat_chunks, CHUNK)
    noidx2 = noidx.reshape(NUM_WORKERS, num_chunks, CHUNK)

    mesh = plsc.VectorSubcoreMesh(core_axis_name="c", subcore_axis_name="s")

    @functools.partial(
        pl.kernel,
        mesh=mesh,
        compiler_params=pltpu.CompilerParams(use_tc_tiling_on_sc=False),
        out_type=jax.ShapeDtypeStruct((out_rows, D), jnp.float32),
        scratch_types=[
            pltpu.VMEM((cat_chunks, CHUNK), jnp.int32),
            pltpu.VMEM((cat_chunks, CHUNK), jnp.int32),
            pltpu.VMEM((num_chunks, CHUNK), jnp.int32),
            pltpu.VMEM((GROUP, CHUNK, D), jnp.float32),
            pltpu.SemaphoreType.DMA,
            pltpu.SemaphoreType.DMA,
        ],
    )
    def k(tbl_hbm, gidx_hbm, oidx_hbm, num_hbm, noidx_hbm, out_hbm,
          gidx_v, oidx_v, noidx_v, rows_v, gsem, ssem):
        wid = lax.axis_index("s") * 2 + lax.axis_index("c")
        num_base = wid * num_per_w

        pltpu.async_copy(gidx_hbm.at[wid], gidx_v, gsem)
        pltpu.async_copy(oidx_hbm.at[wid], oidx_v, ssem)
        pltpu.async_copy(noidx_hbm.at[wid], noidx_v, gsem)
        pltpu.make_async_copy(gidx_hbm.at[wid], gidx_v, gsem).wait()
        pltpu.make_async_copy(noidx_hbm.at[wid], noidx_v, gsem).wait()
        pltpu.make_async_copy(oidx_hbm.at[wid], oidx_v, ssem).wait()

        @pl.loop(0, cat_chunks, step=GROUP)
        def _(j0):
            gathers = [
                pltpu.async_copy(
                    tbl_hbm.at[gidx_v.at[j0 + i]], rows_v.at[i], gsem)
                for i in range(GROUP)
            ]
            scatters = []
            for i in range(GROUP):
                gathers[i].wait()
                scatters.append(pltpu.async_copy(
                    rows_v.at[i], out_hbm.at[oidx_v.at[j0 + i]], ssem))
            for s in scatters:
                s.wait()

        @pl.loop(0, num_chunks)
        def _(j):
            start = num_base + j * CHUNK
            pltpu.sync_copy(num_hbm.at[pl.ds(start, CHUNK)], rows_v.at[0])
            pltpu.sync_copy(rows_v.at[0], out_hbm.at[noidx_v.at[j]])

    return k(tbl_flat, gidx3, oidx3, num_token, noidx2)


def kernel(X_num, X_cat, W_num, b_num, tables):
    B, _ = X_num.shape
    NC = X_cat.shape[1]
    V = tables.shape[1]
    D = tables.shape[2]
    T = NC + 1

    num_token = _num_token_tc(X_num, W_num.T, b_num[None, :])

    x_cat = X_cat.astype(jnp.int32)
    field_off = (jnp.arange(NC, dtype=jnp.int32) * V)[None, :]
    gidx = (x_cat + field_off).reshape(B * NC)
    row_base = (jnp.arange(B, dtype=jnp.int32) * T)[:, None]
    oidx = (row_base + 1 + jnp.arange(NC, dtype=jnp.int32)[None, :]).reshape(B * NC)
    noidx = jnp.arange(B, dtype=jnp.int32) * T

    tbl_flat = tables.reshape(NC * V, D)
    out_flat = _sc_tokens(tbl_flat, gidx, oidx, num_token, noidx, B * T)
    return out_flat.reshape(B, T, D)


# speedup vs baseline: 1.5718x; 1.4085x over previous
import dataclasses
import functools

import jax
import jax.numpy as jnp
from jax import lax
from jax.experimental import pallas as pl
from jax.experimental.pallas import tpu as pltpu
from jax.experimental.pallas import tpu_sc as plsc

NUM_WORKERS = 32
BCHUNK = 128

_SC_PARAMS = pltpu.CompilerParams(use_tc_tiling_on_sc=True)
if "needs_layout_passes" in pltpu.CompilerParams.__dataclass_fields__:
    _SC_PARAMS = dataclasses.replace(_SC_PARAMS, needs_layout_passes=False)


def _num_planes_tc(W_num, XnT, b_num):
    D, K = W_num.shape
    B = XnT.shape[1]
    Bb = 2048

    def body(w_ref, x_ref, b_ref, o_ref):
        o_ref[...] = (
            jnp.dot(w_ref[...], x_ref[...], preferred_element_type=jnp.float32)
            + b_ref[...]
        )

    return pl.pallas_call(
        body,
        grid=(B // Bb,),
        in_specs=[
            pl.BlockSpec((D, K), lambda i: (0, 0)),
            pl.BlockSpec((K, Bb), lambda i: (0, i)),
            pl.BlockSpec((D, 1), lambda i: (0, 0)),
        ],
        out_specs=pl.BlockSpec((D, Bb), lambda i: (0, i)),
        out_shape=jax.ShapeDtypeStruct((D, B), jnp.float32),
    )(W_num, XnT, b_num[:, None])


def _sc_planes(tblF, xT, numT, NC, V, D, B):
    T = NC + 1
    b_per_w = B // NUM_WORKERS
    n_chunks = b_per_w // BCHUNK

    mesh = plsc.VectorSubcoreMesh(core_axis_name="c", subcore_axis_name="s")

    @functools.partial(
        pl.kernel,
        mesh=mesh,
        compiler_params=_SC_PARAMS,
        out_type=jax.ShapeDtypeStruct((T * D, B), jnp.float32),
        scratch_types=[
            pltpu.VMEM((NC, BCHUNK), jnp.int32),
            pltpu.VMEM((2, D, BCHUNK), jnp.int32),
            pltpu.VMEM((2, D, BCHUNK), jnp.float32),
            pltpu.SemaphoreType.DMA,
            pltpu.SemaphoreType.DMA,
        ],
    )
    def k(tbl_hbm, xT_hbm, numT_hbm, out_hbm, xv, eidx, dbuf, gsem, ssem):
        wid = lax.axis_index("s") * 2 + lax.axis_index("c")
        iota16 = lax.iota(jnp.int32, 16)

        def build_eidx(t, buf):
            @pl.loop(0, BCHUNK // 16)
            def _(g):
                v16 = xv[t, pl.ds(g * 16, 16)]
                for c in range(D):
                    eidx[buf, c, pl.ds(g * 16, 16)] = v16 + ((t * D + c) * V)

        @pl.loop(0, n_chunks)
        def _(jb):
            b0 = wid * b_per_w + jb * BCHUNK
            pltpu.sync_copy(xT_hbm.at[:, pl.ds(b0, BCHUNK)], xv)
            pltpu.sync_copy(numT_hbm.at[:, pl.ds(b0, BCHUNK)], dbuf.at[0])
            pltpu.sync_copy(dbuf.at[0], out_hbm.at[pl.ds(0, D), pl.ds(b0, BCHUNK)])

            def start_gathers(buf):
                return [
                    pltpu.async_copy(
                        tbl_hbm.at[eidx.at[buf, c]], dbuf.at[buf, c], gsem)
                    for c in range(D)
                ]

            build_eidx(0, 0)
            hs = start_gathers(0)
            for t in range(NC):
                nbuf = (t + 1) % 2
                cbuf = t % 2
                if t + 1 < NC:
                    build_eidx(t + 1, nbuf)
                    hn = start_gathers(nbuf)
                for h in hs:
                    h.wait()
                pltpu.sync_copy(
                    dbuf.at[cbuf],
                    out_hbm.at[pl.ds((t + 1) * D, D), pl.ds(b0, BCHUNK)])
                if t + 1 < NC:
                    hs = hn

    return k(tblF, xT, numT)


def kernel(X_num, X_cat, W_num, b_num, tables):
    B, _ = X_num.shape
    NC = X_cat.shape[1]
    V = tables.shape[1]
    D = tables.shape[2]
    T = NC + 1

    numT = _num_planes_tc(W_num, X_num.T, b_num)
    xT = X_cat.astype(jnp.int32).T
    tblF = jnp.swapaxes(tables, 1, 2).reshape(NC * D * V)

    out_planes = _sc_planes(tblF, xT, numT, NC, V, D, B)
    return out_planes.reshape(T, D, B).transpose(2, 0, 1)


# speedup vs baseline: 1.5762x; 1.0028x over previous
import dataclasses
import functools

import jax
import jax.numpy as jnp
from jax import lax
from jax.experimental import pallas as pl
from jax.experimental.pallas import tpu as pltpu
from jax.experimental.pallas import tpu_sc as plsc

NUM_WORKERS = 32
BCHUNK = 128

_SC_PARAMS = pltpu.CompilerParams(use_tc_tiling_on_sc=True)
if "needs_layout_passes" in pltpu.CompilerParams.__dataclass_fields__:
    _SC_PARAMS = dataclasses.replace(_SC_PARAMS, needs_layout_passes=False)


def _num_planes_tc(W_num, XnT, b_num):
    D, K = W_num.shape
    B = XnT.shape[1]
    Bb = 2048

    def body(w_ref, x_ref, b_ref, o_ref):
        o_ref[...] = (
            jnp.dot(w_ref[...], x_ref[...], preferred_element_type=jnp.float32)
            + b_ref[...]
        )

    return pl.pallas_call(
        body,
        grid=(B // Bb,),
        in_specs=[
            pl.BlockSpec((D, K), lambda i: (0, 0)),
            pl.BlockSpec((K, Bb), lambda i: (0, i)),
            pl.BlockSpec((D, 1), lambda i: (0, 0)),
        ],
        out_specs=pl.BlockSpec((D, Bb), lambda i: (0, i)),
        out_shape=jax.ShapeDtypeStruct((D, B), jnp.float32),
    )(W_num, XnT, b_num[:, None])


def _sc_planes(tblF, xT, numT, NC, V, D, B):
    T = NC + 1
    b_per_w = B // NUM_WORKERS
    n_chunks = b_per_w // BCHUNK

    mesh = plsc.VectorSubcoreMesh(core_axis_name="c", subcore_axis_name="s")

    @functools.partial(
        pl.kernel,
        mesh=mesh,
        compiler_params=_SC_PARAMS,
        out_type=jax.ShapeDtypeStruct((T * D, B), jnp.float32),
        scratch_types=[
            pltpu.VMEM((NC, BCHUNK), jnp.int32),
            pltpu.VMEM((2, D, BCHUNK), jnp.int32),
            pltpu.VMEM((2, D, BCHUNK), jnp.float32),
            pltpu.SemaphoreType.DMA,
            pltpu.SemaphoreType.DMA,
        ],
    )
    def k(tbl_hbm, xT_hbm, numT_hbm, out_hbm, xv, eidx, dbuf, gsem, ssem):
        wid = lax.axis_index("s") * 2 + lax.axis_index("c")
        iota16 = lax.iota(jnp.int32, 16)

        def build_eidx(t, buf):
            @pl.loop(0, BCHUNK // 16)
            def _(g):
                v16 = xv[t, pl.ds(g * 16, 16)]
                for c in range(D):
                    eidx[buf, c, pl.ds(g * 16, 16)] = v16 + ((t * D + c) * V)

        @pl.loop(0, n_chunks)
        def _(jb):
            b0 = wid * b_per_w + jb * BCHUNK
            pltpu.sync_copy(xT_hbm.at[:, pl.ds(b0, BCHUNK)], xv)
            pltpu.sync_copy(numT_hbm.at[:, pl.ds(b0, BCHUNK)], dbuf.at[0])
            pltpu.sync_copy(dbuf.at[0], out_hbm.at[pl.ds(0, D), pl.ds(b0, BCHUNK)])

            def start_gathers(buf):
                return [
                    pltpu.async_copy(
                        tbl_hbm.at[eidx.at[buf, c]], dbuf.at[buf, c], gsem)
                    for c in range(D)
                ]

            def drain_write():
                pltpu.make_async_copy(
                    dbuf.at[0],
                    out_hbm.at[pl.ds(0, D), pl.ds(b0, BCHUNK)], ssem).wait()

            build_eidx(0, 0)
            hs = start_gathers(0)
            for t in range(NC):
                nbuf = (t + 1) % 2
                cbuf = t % 2
                if t + 1 < NC:
                    build_eidx(t + 1, nbuf)
                    if t >= 1:
                        drain_write()
                    hn = start_gathers(nbuf)
                for h in hs:
                    h.wait()
                pltpu.async_copy(
                    dbuf.at[cbuf],
                    out_hbm.at[pl.ds((t + 1) * D, D), pl.ds(b0, BCHUNK)], ssem)
                if t + 1 < NC:
                    hs = hn
            drain_write()
            drain_write()

    return k(tblF, xT, numT)


def kernel(X_num, X_cat, W_num, b_num, tables):
    B, _ = X_num.shape
    NC = X_cat.shape[1]
    V = tables.shape[1]
    D = tables.shape[2]
    T = NC + 1

    numT = _num_planes_tc(W_num, X_num.T, b_num)
    xT = X_cat.astype(jnp.int32).T
    tblF = jnp.swapaxes(tables, 1, 2).reshape(NC * D * V)

    out_planes = _sc_planes(tblF, xT, numT, NC, V, D, B)
    return out_planes.reshape(T, D, B).transpose(2, 0, 1)


# speedup vs baseline: 1.8226x; 1.1563x over previous
import dataclasses
import functools

import jax
import jax.numpy as jnp
from jax import lax
from jax.experimental import pallas as pl
from jax.experimental.pallas import tpu as pltpu
from jax.experimental.pallas import tpu_sc as plsc

NUM_WORKERS = 32
BCHUNK = 128

_SC_PARAMS = pltpu.CompilerParams(use_tc_tiling_on_sc=True)
if "needs_layout_passes" in pltpu.CompilerParams.__dataclass_fields__:
    _SC_PARAMS = dataclasses.replace(_SC_PARAMS, needs_layout_passes=False)


def _num_planes_tc(W_num, XnT, b_num):
    D, K = W_num.shape
    B = XnT.shape[1]
    Bb = 2048

    def body(w_ref, x_ref, b_ref, o_ref):
        o_ref[...] = (
            jnp.dot(w_ref[...], x_ref[...], preferred_element_type=jnp.float32)
            + b_ref[...]
        )

    return pl.pallas_call(
        body,
        grid=(B // Bb,),
        in_specs=[
            pl.BlockSpec((D, K), lambda i: (0, 0)),
            pl.BlockSpec((K, Bb), lambda i: (0, i)),
            pl.BlockSpec((D, 1), lambda i: (0, 0)),
        ],
        out_specs=pl.BlockSpec((D, Bb), lambda i: (0, i)),
        out_shape=jax.ShapeDtypeStruct((D, B), jnp.float32),
    )(W_num, XnT, b_num[:, None])


def _sc_detile(t3, tails, NC, V, D, V2):
    n_blocks = NC * D // 8
    WMAIN = 6400
    WINS = [(k * WMAIN, WMAIN) for k in range(V2 // WMAIN)]
    if V2 % WMAIN:
        WINS.append(((V2 // WMAIN) * WMAIN, V2 % WMAIN))
    main_sz = NC * D * V2
    tail_sz = NC * D * (V - V2)
    half_tail = tail_sz // 2

    mesh = plsc.VectorSubcoreMesh(core_axis_name="c", subcore_axis_name="s")

    @functools.partial(
        pl.kernel,
        mesh=mesh,
        compiler_params=_SC_PARAMS,
        out_type=jax.ShapeDtypeStruct((NC * D * V,), jnp.float32),
        scratch_types=[
            pltpu.VMEM((2, 8, WMAIN), jnp.float32),
            pltpu.VMEM((half_tail,), jnp.float32),
            pltpu.SemaphoreType.DMA,
            pltpu.SemaphoreType.DMA,
        ],
    )
    def kd(t3_hbm, tails_hbm, outF_hbm, buf, tbuf, rsem, wsem):
        wid = lax.axis_index("s") * 2 + lax.axis_index("c")
        nb = jnp.where(wid < n_blocks % NUM_WORKERS,
                       n_blocks // NUM_WORKERS + 1,
                       n_blocks // NUM_WORKERS)

        @pl.when(wid < 2)
        def _():
            off = wid * half_tail
            pltpu.sync_copy(tails_hbm.at[pl.ds(off, half_tail)], tbuf)
            pltpu.sync_copy(tbuf, outF_hbm.at[pl.ds(main_sz + off, half_tail)])

        @pl.loop(0, nb)
        def _(kb):
            blk = wid + NUM_WORKERS * kb
            t = lax.shift_right_logical(blk, 2)
            c0 = jnp.bitwise_and(blk, 3) * 8
            rbase = blk * 8 * V2

            whs = {}
            for j, (v0, wl) in enumerate(WINS):
                b = j & 1
                if j >= 2:
                    for h in whs[j - 2]:
                        h.wait()
                pltpu.async_copy(
                    t3_hbm.at[t, pl.ds(c0, 8), pl.ds(v0, wl)],
                    buf.at[b, :, pl.ds(0, wl)], rsem).wait()
                whs[j] = [
                    pltpu.async_copy(
                        buf.at[b, jj, pl.ds(0, wl)],
                        outF_hbm.at[pl.ds(rbase + jj * V2 + v0, wl)], wsem)
                    for jj in range(8)
                ]
            for j in (len(WINS) - 2, len(WINS) - 1):
                for h in whs[j]:
                    h.wait()

    return kd(t3, tails)


def _sc_planes(tblF, xT, numT, NC, V, D, B, V2):
    T = NC + 1
    b_per_w = B // NUM_WORKERS
    n_chunks = b_per_w // BCHUNK

    mesh = plsc.VectorSubcoreMesh(core_axis_name="c", subcore_axis_name="s")

    @functools.partial(
        pl.kernel,
        mesh=mesh,
        compiler_params=_SC_PARAMS,
        out_type=jax.ShapeDtypeStruct((T * D, B), jnp.float32),
        scratch_types=[
            pltpu.VMEM((NC, BCHUNK), jnp.int32),
            pltpu.VMEM((2, D, BCHUNK), jnp.int32),
            pltpu.VMEM((2, D, BCHUNK), jnp.float32),
            pltpu.SemaphoreType.DMA,
            pltpu.SemaphoreType.DMA,
        ],
    )
    def k(tbl_hbm, xT_hbm, numT_hbm, out_hbm, xv, eidx, dbuf, gsem, ssem):
        wid = lax.axis_index("s") * 2 + lax.axis_index("c")
        iota16 = lax.iota(jnp.int32, 16)

        def build_eidx(t, buf):
            main_sz = NC * D * V2

            @pl.loop(0, BCHUNK // 16)
            def _(g):
                v16 = xv[t, pl.ds(g * 16, 16)]
                in_main = v16 < V2
                for c in range(D):
                    r = t * D + c
                    idx_main = v16 + (r * V2)
                    idx_tail = v16 + (main_sz + r * (V - V2) - V2)
                    eidx[buf, c, pl.ds(g * 16, 16)] = jnp.where(
                        in_main, idx_main, idx_tail)

        @pl.loop(0, n_chunks)
        def _(jb):
            b0 = wid * b_per_w + jb * BCHUNK
            pltpu.sync_copy(xT_hbm.at[:, pl.ds(b0, BCHUNK)], xv)
            pltpu.sync_copy(numT_hbm.at[:, pl.ds(b0, BCHUNK)], dbuf.at[0])
            pltpu.sync_copy(dbuf.at[0], out_hbm.at[pl.ds(0, D), pl.ds(b0, BCHUNK)])

            def start_gathers(buf):
                return [
                    pltpu.async_copy(
                        tbl_hbm.at[eidx.at[buf, c]], dbuf.at[buf, c], gsem)
                    for c in range(D)
                ]

            def drain_write():
                pltpu.make_async_copy(
                    dbuf.at[0],
                    out_hbm.at[pl.ds(0, D), pl.ds(b0, BCHUNK)], ssem).wait()

            build_eidx(0, 0)
            hs = start_gathers(0)
            for t in range(NC):
                nbuf = (t + 1) % 2
                cbuf = t % 2
                if t + 1 < NC:
                    build_eidx(t + 1, nbuf)
                    if t >= 1:
                        drain_write()
                    hn = start_gathers(nbuf)
                for h in hs:
                    h.wait()
                pltpu.async_copy(
                    dbuf.at[cbuf],
                    out_hbm.at[pl.ds((t + 1) * D, D), pl.ds(b0, BCHUNK)], ssem)
                if t + 1 < NC:
                    hs = hn
            drain_write()
            drain_write()

    return k(tblF, xT, numT)


def kernel(X_num, X_cat, W_num, b_num, tables):
    B, _ = X_num.shape
    NC = X_cat.shape[1]
    V = tables.shape[1]
    D = tables.shape[2]
    T = NC + 1

    numT = _num_planes_tc(W_num, X_num.T, b_num)
    xT = X_cat.astype(jnp.int32).T
    V2 = (V // 128) * 128
    tails = jnp.swapaxes(tables[:, V2:, :], 1, 2).reshape(NC * D * (V - V2))
    tblF = _sc_detile(jnp.swapaxes(tables, 1, 2), tails, NC, V, D, V2)

    out_planes = _sc_planes(tblF, xT, numT, NC, V, D, B, V2)
    return out_planes.reshape(T, D, B).transpose(2, 0, 1)


# speedup vs baseline: 1.8229x; 1.0002x over previous
import dataclasses
import functools

import jax
import jax.numpy as jnp
from jax import lax
from jax.experimental import pallas as pl
from jax.experimental.pallas import tpu as pltpu
from jax.experimental.pallas import tpu_sc as plsc

NUM_WORKERS = 32
BCHUNK = 128

_SC_PARAMS = pltpu.CompilerParams(use_tc_tiling_on_sc=True)
if "needs_layout_passes" in pltpu.CompilerParams.__dataclass_fields__:
    _SC_PARAMS = dataclasses.replace(_SC_PARAMS, needs_layout_passes=False)


def _num_planes_tc(W_num, XnT, b_num):
    D, K = W_num.shape
    B = XnT.shape[1]
    Bb = 2048

    def body(w_ref, x_ref, b_ref, o_ref):
        o_ref[...] = (
            jnp.dot(w_ref[...], x_ref[...], preferred_element_type=jnp.float32)
            + b_ref[...]
        )

    return pl.pallas_call(
        body,
        grid=(B // Bb,),
        in_specs=[
            pl.BlockSpec((D, K), lambda i: (0, 0)),
            pl.BlockSpec((K, Bb), lambda i: (0, i)),
            pl.BlockSpec((D, 1), lambda i: (0, 0)),
        ],
        out_specs=pl.BlockSpec((D, Bb), lambda i: (0, i)),
        out_shape=jax.ShapeDtypeStruct((D, B), jnp.float32),
    )(W_num, XnT, b_num[:, None])


def _sc_detile(t3, tails, NC, V, D, V2):
    n_blocks = NC * D // 8
    WMAIN = 6400
    WINS = [(k * WMAIN, WMAIN) for k in range(V2 // WMAIN)]
    if V2 % WMAIN:
        WINS.append(((V2 // WMAIN) * WMAIN, V2 % WMAIN))
    main_sz = NC * D * V2
    tail_sz = NC * D * (V - V2)
    half_tail = tail_sz // 2

    mesh = plsc.VectorSubcoreMesh(core_axis_name="c", subcore_axis_name="s")

    @functools.partial(
        pl.kernel,
        mesh=mesh,
        compiler_params=_SC_PARAMS,
        out_type=jax.ShapeDtypeStruct((NC * D * V,), jnp.float32),
        scratch_types=[
            pltpu.VMEM((2, 8, WMAIN), jnp.float32),
            pltpu.VMEM((half_tail,), jnp.float32),
            pltpu.SemaphoreType.DMA,
            pltpu.SemaphoreType.DMA,
        ],
    )
    def kd(t3_hbm, tails_hbm, outF_hbm, buf, tbuf, rsem, wsem):
        wid = lax.axis_index("s") * 2 + lax.axis_index("c")
        nb = jnp.where(wid < n_blocks % NUM_WORKERS,
                       n_blocks // NUM_WORKERS + 1,
                       n_blocks // NUM_WORKERS)

        @pl.when(wid < 2)
        def _():
            off = wid * half_tail
            pltpu.sync_copy(tails_hbm.at[pl.ds(off, half_tail)], tbuf)
            pltpu.sync_copy(tbuf, outF_hbm.at[pl.ds(main_sz + off, half_tail)])

        @pl.loop(0, nb)
        def _(kb):
            blk = wid + NUM_WORKERS * kb
            t = lax.shift_right_logical(blk, 2)
            c0 = jnp.bitwise_and(blk, 3) * 8
            rbase = blk * 8 * V2

            whs = {}
            for j, (v0, wl) in enumerate(WINS):
                b = j & 1
                if j >= 2:
                    for h in whs[j - 2]:
                        h.wait()
                pltpu.async_copy(
                    t3_hbm.at[t, pl.ds(c0, 8), pl.ds(v0, wl)],
                    buf.at[b, :, pl.ds(0, wl)], rsem).wait()
                whs[j] = [
                    pltpu.async_copy(
                        buf.at[b, jj, pl.ds(0, wl)],
                        outF_hbm.at[pl.ds(rbase + jj * V2 + v0, wl)], wsem)
                    for jj in range(8)
                ]
            for j in (len(WINS) - 2, len(WINS) - 1):
                for h in whs[j]:
                    h.wait()

    return kd(t3, tails)


def _sc_planes(tblF, xT, numT, NC, V, D, B, V2):
    T = NC + 1
    b_per_w = B // NUM_WORKERS
    n_chunks = b_per_w // BCHUNK

    mesh = plsc.VectorSubcoreMesh(core_axis_name="c", subcore_axis_name="s")

    @functools.partial(
        pl.kernel,
        mesh=mesh,
        compiler_params=_SC_PARAMS,
        out_type=jax.ShapeDtypeStruct((T * D, B), jnp.float32),
        scratch_types=[
            pltpu.VMEM((NC, BCHUNK), jnp.int32),
            pltpu.VMEM((2, D, BCHUNK), jnp.int32),
            pltpu.VMEM((2, D, BCHUNK), jnp.float32),
            pltpu.SemaphoreType.DMA,
            pltpu.SemaphoreType.DMA,
        ],
    )
    def k(tbl_hbm, xT_hbm, numT_hbm, out_hbm, xv, eidx, dbuf, gsem, ssem):
        wid = lax.axis_index("s") * 2 + lax.axis_index("c")

        def build_eidx(t, buf):
            main_sz = NC * D * V2

            @pl.loop(0, BCHUNK // 16)
            def _(g):
                v16 = xv[t, pl.ds(g * 16, 16)]
                in_main = v16 < V2
                for c in range(D):
                    r = t * D + c
                    idx_main = v16 + (r * V2)
                    idx_tail = v16 + (main_sz + r * (V - V2) - V2)
                    eidx[buf, c, pl.ds(g * 16, 16)] = jnp.where(
                        in_main, idx_main, idx_tail)

        @pl.loop(0, n_chunks)
        def _(jb):
            b0 = wid * b_per_w + jb * BCHUNK
            pltpu.sync_copy(xT_hbm.at[:, pl.ds(b0, BCHUNK)], xv)
            pltpu.sync_copy(numT_hbm.at[:, pl.ds(b0, BCHUNK)], dbuf.at[0])
            pltpu.sync_copy(dbuf.at[0], out_hbm.at[pl.ds(0, D), pl.ds(b0, BCHUNK)])

            def start_gathers(buf):
                return [
                    pltpu.async_copy(
                        tbl_hbm.at[eidx.at[buf, c]], dbuf.at[buf, c], gsem)
                    for c in range(D)
                ]

            def drain_write():
                pltpu.make_async_copy(
                    dbuf.at[0],
                    out_hbm.at[pl.ds(0, D), pl.ds(b0, BCHUNK)], ssem).wait()

            build_eidx(0, 0)
            hs = start_gathers(0)
            for t in range(NC):
                nbuf = (t + 1) % 2
                cbuf = t % 2
                if t + 1 < NC:
                    build_eidx(t + 1, nbuf)
                    if t >= 1:
                        drain_write()
                    hn = start_gathers(nbuf)
                for h in hs:
                    h.wait()
                pltpu.async_copy(
                    dbuf.at[cbuf],
                    out_hbm.at[pl.ds((t + 1) * D, D), pl.ds(b0, BCHUNK)], ssem)
                if t + 1 < NC:
                    hs = hn
            drain_write()
            drain_write()

    return k(tblF, xT, numT)


def kernel(X_num, X_cat, W_num, b_num, tables):
    B, _ = X_num.shape
    NC = X_cat.shape[1]
    V = tables.shape[1]
    D = tables.shape[2]
    T = NC + 1

    numT = _num_planes_tc(W_num, X_num.T, b_num)
    xT = X_cat.astype(jnp.int32).T
    V2 = (V // 128) * 128
    tails = jnp.swapaxes(tables[:, V2:, :], 1, 2).reshape(NC * D * (V - V2))
    tblF = _sc_detile(jnp.swapaxes(tables, 1, 2), tails, NC, V, D, V2)

    out_planes = _sc_planes(tblF, xT, numT, NC, V, D, B, V2)
    return out_planes.reshape(T, D, B).transpose(2, 0, 1)
